# baseline (speedup 1.0000x reference)
import functools

import jax
import jax.numpy as jnp
from jax import lax
from jax.experimental import pallas as pl
from jax.experimental.pallas import tpu as pltpu
from jax.experimental.pallas import tpu_sc as plsc

_DIM = 32
_HALF = 512


def _sc_embed(ids_native, table):
    n = ids_native.shape[0]
    info = plsc.get_sparse_core_info()
    nc, ns, lanes = info.num_cores, info.num_subcores, info.num_lanes
    nw = nc * ns
    assert n % nw == 0
    per_w = n // nw
    assert per_w % (2 * _HALF) == 0
    n_half = per_w // _HALF
    n_iter = n_half // 2
    units_per_w = per_w // 1024
    groups = _HALF // lanes

    @functools.partial(
        pl.kernel,
        mesh=plsc.VectorSubcoreMesh(core_axis_name="c", subcore_axis_name="s"),
        out_type=jax.ShapeDtypeStruct((n * _DIM,), jnp.float32),
        compiler_params=pltpu.CompilerParams(
            use_tc_tiling_on_sc=False, needs_layout_passes=False
        ),
        scratch_types=[
            pltpu.VMEM((per_w,), jnp.int32),
            pltpu.VMEM((_HALF, _DIM), jnp.float32),
            pltpu.VMEM((_HALF, _DIM), jnp.float32),
            pltpu.VMEM((_HALF * _DIM,), jnp.float32),
            pltpu.VMEM((_HALF * _DIM,), jnp.float32),
            pltpu.SemaphoreType.DMA,
            pltpu.SemaphoreType.DMA,
            pltpu.SemaphoreType.DMA,
            pltpu.SemaphoreType.DMA,
        ],
    )
    def run(table_hbm, ids_hbm, out_hbm, idx_all, rows_a, rows_b,
            trans_a, trans_b, ga, gb, oa, ob):
        wid = lax.axis_index("s") * nc + lax.axis_index("c")
        kbase = wid * per_w
        u0 = wid * units_per_w

        pltpu.sync_copy(ids_hbm.at[pl.ds(kbase, per_w)], idx_all)

        iota16 = lax.iota(jnp.int32, lanes)
        bufs = ((rows_a, trans_a, ga, oa), (rows_b, trans_b, gb, ob))

        def gather(h, rows, gsem):
            pltpu.async_copy(
                table_hbm.at[idx_all.at[pl.ds(h * _HALF, _HALF)]], rows, gsem
            )

        gather(0, rows_a, ga)
        gather(1, rows_b, gb)

        def body(i2, _):
            for p, (rows, trans, gsem, osem) in enumerate(bufs):
                h = 2 * i2 + p
                u = u0 + i2
                b1_hi = lax.shift_right_logical(u, 5)
                b0_hi = lax.bitwise_and(u, 31)
                hloc = h * _HALF

                pltpu.make_async_copy(
                    table_hbm.at[pl.ds(0, _HALF)], rows, gsem
                ).wait()

                def min_body(g, a):
                    return jnp.minimum(
                        a, idx_all[pl.ds(hloc + g * lanes, lanes)]
                    )

                accv = lax.fori_loop(
                    0, groups, min_body, jnp.full((lanes,), 1, jnp.int32)
                )

                @pl.when(jnp.min(accv) == 0)
                def _():
                    def scan_body(g, _):
                        v = idx_all[pl.ds(hloc + g * lanes, lanes)]
                        m = v == 0
                        cnt = jnp.sum(jnp.where(m, 1, 0))

                        @pl.when(cnt > 0)
                        def _():
                            rvec = g * lanes + iota16
                            zero = jnp.zeros((lanes,), jnp.float32)
                            for j in range(_DIM):
                                cvec = jnp.full((lanes,), j, jnp.int32)
                                plsc.store_scatter(
                                    rows, [rvec, cvec], zero, mask=m
                                )

                        return 0

                    lax.fori_loop(0, groups, scan_body, 0)

                @pl.when(i2 >= 1)
                def _():
                    pltpu.make_async_copy(
                        out_hbm.at[pl.ds(0, _HALF * _DIM)], trans, osem
                    ).wait()

                def tbody(j, _):
                    b1l = lax.shift_right_logical(j, 5)
                    c = lax.bitwise_and(j, 31)
                    chi = lax.shift_right_logical(c, 3)
                    clo = lax.bitwise_and(c, 7)
                    obase = b1l * 4096 + chi * 1024 + clo * 128
                    rbase = b1l * 128
                    cvec = jnp.full((lanes,), c, jnp.int32)
                    for k in range(8):
                        rvec = rbase + k * lanes + iota16
                        v = plsc.load_gather(rows, [rvec, cvec])
                        trans[pl.ds(obase + k * lanes, lanes)] = v
                    return 0

                lax.fori_loop(0, 4 * _DIM, tbody, 0)

                for b1l in range(4):
                    b1 = b1_hi * 8 + p * 4 + b1l
                    for chi in range(4):
                        pltpu.async_copy(
                            trans.at[pl.ds((b1l * 4 + chi) * 1024, 1024)],
                            out_hbm.at[
                                pl.ds(
                                    b1 * 131072 + chi * 32768 + b0_hi * 1024,
                                    1024,
                                )
                            ],
                            osem,
                        )

                @pl.when(i2 < n_iter - 1)
                def _():
                    gather(h + 2, rows, gsem)

            return 0

        lax.fori_loop(0, n_iter, body, 0)

        for p, (rows, trans, gsem, osem) in enumerate(bufs):
            pltpu.make_async_copy(
                out_hbm.at[pl.ds(0, _HALF * _DIM)], trans, osem
            ).wait()

    return run(table, ids_native)


def kernel(input_ids, embedding_matrix):
    b, s = input_ids.shape
    ids_native = (
        input_ids.T.astype(jnp.int32)
        .reshape(s // 8, 8, b // 128, 128)
        .transpose(0, 2, 1, 3)
        .reshape(b * s)
    )
    out_flat = _sc_embed(ids_native, embedding_matrix)
    return (
        out_flat.reshape(s, 4, b // 128, 8, 128)
        .transpose(2, 4, 0, 1, 3)
        .reshape(b, s, _DIM)
    )


# speedup vs baseline: 1.2240x; 1.2240x over previous
import functools

import jax
import jax.numpy as jnp
from jax import lax
from jax.experimental import pallas as pl
from jax.experimental.pallas import tpu as pltpu
from jax.experimental.pallas import tpu_sc as plsc

_DIM = 32
_HALF = 512


def _sc_embed(ids_native, table):
    n = ids_native.shape[0]
    info = plsc.get_sparse_core_info()
    nc, ns, lanes = info.num_cores, info.num_subcores, info.num_lanes
    nw = nc * ns
    assert n % nw == 0
    per_w = n // nw
    assert per_w % (2 * _HALF) == 0
    n_half = per_w // _HALF
    n_iter = n_half // 2
    units_per_w = per_w // 1024
    groups = _HALF // lanes

    @functools.partial(
        pl.kernel,
        mesh=plsc.VectorSubcoreMesh(core_axis_name="c", subcore_axis_name="s"),
        out_type=jax.ShapeDtypeStruct((n * _DIM,), jnp.float32),
        compiler_params=pltpu.CompilerParams(
            use_tc_tiling_on_sc=False, needs_layout_passes=False
        ),
        scratch_types=[
            pltpu.VMEM((per_w,), jnp.int32),
            pltpu.VMEM((_HALF, _DIM), jnp.float32),
            pltpu.VMEM((_HALF, _DIM), jnp.float32),
            pltpu.VMEM((_HALF * _DIM,), jnp.float32),
            pltpu.VMEM((_HALF * _DIM,), jnp.float32),
            pltpu.SemaphoreType.DMA,
            pltpu.SemaphoreType.DMA,
            pltpu.SemaphoreType.DMA,
            pltpu.SemaphoreType.DMA,
        ],
    )
    def run(table_hbm, ids_hbm, out_hbm, idx_all, rows_a, rows_b,
            trans_a, trans_b, ga, gb, oa, ob):
        wid = lax.axis_index("s") * nc + lax.axis_index("c")
        kbase = wid * per_w
        u0 = wid * units_per_w

        pltpu.sync_copy(ids_hbm.at[pl.ds(kbase, per_w)], idx_all)

        iota16 = lax.iota(jnp.int32, lanes)
        cmap0 = (
            lax.shift_right_logical(iota16, 3) * 1024
            + lax.bitwise_and(iota16, 7) * 128
        )
        bufs = ((rows_a, trans_a, ga, oa), (rows_b, trans_b, gb, ob))

        def gather(h, rows, gsem):
            pltpu.async_copy(
                table_hbm.at[idx_all.at[pl.ds(h * _HALF, _HALF)]], rows, gsem
            )

        gather(0, rows_a, ga)
        gather(1, rows_b, gb)

        def body(i2, _):
            for p, (rows, trans, gsem, osem) in enumerate(bufs):
                h = 2 * i2 + p
                u = u0 + i2
                b1_hi = lax.shift_right_logical(u, 5)
                b0_hi = lax.bitwise_and(u, 31)
                hloc = h * _HALF

                pltpu.make_async_copy(
                    table_hbm.at[pl.ds(0, _HALF)], rows, gsem
                ).wait()

                def min_body(g, a):
                    return jnp.minimum(
                        a, idx_all[pl.ds(hloc + g * lanes, lanes)]
                    )

                accv = lax.fori_loop(
                    0, groups, min_body, jnp.full((lanes,), 1, jnp.int32)
                )

                @pl.when(jnp.min(accv) == 0)
                def _():
                    def scan_body(g, _):
                        v = idx_all[pl.ds(hloc + g * lanes, lanes)]
                        m = v == 0
                        cnt = jnp.sum(jnp.where(m, 1, 0))

                        @pl.when(cnt > 0)
                        def _():
                            rvec = g * lanes + iota16
                            zero = jnp.zeros((lanes,), jnp.float32)
                            for j in range(_DIM):
                                cvec = jnp.full((lanes,), j, jnp.int32)
                                plsc.store_scatter(
                                    rows, [rvec, cvec], zero, mask=m
                                )

                        return 0

                    lax.fori_loop(0, groups, scan_body, 0)

                @pl.when(i2 >= 1)
                def _():
                    pltpu.make_async_copy(
                        out_hbm.at[pl.ds(0, _HALF * _DIM)], trans, osem
                    ).wait()

                @plsc.parallel_loop(0, _HALF, unroll=8)
                def _(r):
                    b1l = lax.shift_right_logical(r, 7)
                    b0 = lax.bitwise_and(r, 127)
                    base = b1l * 4096 + b0
                    v0 = rows[r, pl.ds(0, lanes)]
                    v1 = rows[r, pl.ds(lanes, lanes)]
                    a0 = cmap0 + base
                    plsc.store_scatter(trans, [a0], v0)
                    plsc.store_scatter(trans, [a0 + 2048], v1)

                for b1l in range(4):
                    b1 = b1_hi * 8 + p * 4 + b1l
                    for chi in range(4):
                        pltpu.async_copy(
                            trans.at[pl.ds((b1l * 4 + chi) * 1024, 1024)],
                            out_hbm.at[
                                pl.ds(
                                    b1 * 131072 + chi * 32768 + b0_hi * 1024,
                                    1024,
                                )
                            ],
                            osem,
                        )

                @pl.when(i2 < n_iter - 1)
                def _():
                    gather(h + 2, rows, gsem)

            return 0

        lax.fori_loop(0, n_iter, body, 0)

        for p, (rows, trans, gsem, osem) in enumerate(bufs):
            pltpu.make_async_copy(
                out_hbm.at[pl.ds(0, _HALF * _DIM)], trans, osem
            ).wait()

    return run(table, ids_native)


def kernel(input_ids, embedding_matrix):
    b, s = input_ids.shape
    ids_native = (
        input_ids.T.astype(jnp.int32)
        .reshape(s // 8, 8, b // 128, 128)
        .transpose(0, 2, 1, 3)
        .reshape(b * s)
    )
    out_flat = _sc_embed(ids_native, embedding_matrix)
    return (
        out_flat.reshape(s, 4, b // 128, 8, 128)
        .transpose(2, 4, 0, 1, 3)
        .reshape(b, s, _DIM)
    )


# speedup vs baseline: 1.2833x; 1.0485x over previous
import functools

import jax
import jax.numpy as jnp
from jax import lax
from jax.experimental import pallas as pl
from jax.experimental.pallas import tpu as pltpu
from jax.experimental.pallas import tpu_sc as plsc

_DIM = 32
_HALF = 512


def _sc_embed(ids_native, table):
    n = ids_native.shape[0]
    info = plsc.get_sparse_core_info()
    nc, ns, lanes = info.num_cores, info.num_subcores, info.num_lanes
    nw = nc * ns
    assert n % nw == 0
    per_w = n // nw
    assert per_w % (2 * _HALF) == 0
    n_half = per_w // _HALF
    n_iter = n_half // 2
    units_per_w = per_w // 1024
    groups = _HALF // lanes

    @functools.partial(
        pl.kernel,
        mesh=plsc.VectorSubcoreMesh(core_axis_name="c", subcore_axis_name="s"),
        out_type=jax.ShapeDtypeStruct((200, 4, n * _DIM // 800), jnp.float32),
        compiler_params=pltpu.CompilerParams(
            use_tc_tiling_on_sc=False, needs_layout_passes=False
        ),
        scratch_types=[
            pltpu.VMEM((per_w,), jnp.int32),
            pltpu.VMEM((_HALF, _DIM), jnp.float32),
            pltpu.VMEM((_HALF, _DIM), jnp.float32),
            pltpu.VMEM((4, 4, 1024), jnp.float32),
            pltpu.VMEM((4, 4, 1024), jnp.float32),
            pltpu.SemaphoreType.DMA,
            pltpu.SemaphoreType.DMA,
            pltpu.SemaphoreType.DMA,
            pltpu.SemaphoreType.DMA,
        ],
    )
    def run(table_hbm, ids_hbm, out_hbm, idx_all, rows_a, rows_b,
            trans_a, trans_b, ga, gb, oa, ob):
        wid = lax.axis_index("s") * nc + lax.axis_index("c")
        kbase = wid * per_w
        u0 = wid * units_per_w

        pltpu.sync_copy(ids_hbm.at[pl.ds(kbase, per_w)], idx_all)

        iota16 = lax.iota(jnp.int32, lanes)
        chivec = lax.shift_right_logical(iota16, 3)
        clovec = lax.bitwise_and(iota16, 7) * 128
        bufs = ((rows_a, trans_a, ga, oa), (rows_b, trans_b, gb, ob))

        def gather(h, rows, gsem):
            pltpu.async_copy(
                table_hbm.at[idx_all.at[pl.ds(h * _HALF, _HALF)]], rows, gsem
            )

        gather(0, rows_a, ga)
        gather(1, rows_b, gb)

        def body(i2, _):
            for p, (rows, trans, gsem, osem) in enumerate(bufs):
                h = 2 * i2 + p
                u = u0 + i2
                b1_hi = lax.shift_right_logical(u, 5)
                b0_hi = lax.bitwise_and(u, 31)
                hloc = h * _HALF

                pltpu.make_async_copy(
                    table_hbm.at[pl.ds(0, _HALF)], rows, gsem
                ).wait()

                def min_body(g, a):
                    return jnp.minimum(
                        a, idx_all[pl.ds(hloc + g * lanes, lanes)]
                    )

                accv = lax.fori_loop(
                    0, groups, min_body, jnp.full((lanes,), 1, jnp.int32)
                )

                @pl.when(jnp.min(accv) == 0)
                def _():
                    def scan_body(g, _):
                        v = idx_all[pl.ds(hloc + g * lanes, lanes)]
                        m = v == 0
                        cnt = jnp.sum(jnp.where(m, 1, 0))

                        @pl.when(cnt > 0)
                        def _():
                            rvec = g * lanes + iota16
                            zero = jnp.zeros((lanes,), jnp.float32)
                            for j in range(_DIM):
                                cvec = jnp.full((lanes,), j, jnp.int32)
                                plsc.store_scatter(
                                    rows, [rvec, cvec], zero, mask=m
                                )

                        return 0

                    lax.fori_loop(0, groups, scan_body, 0)

                @pl.when(i2 >= 1)
                def _():
                    pltpu.make_async_copy(
                        out_hbm.at[pl.ds(0, 4), :, pl.ds(0, 1024)],
                        trans,
                        osem,
                    ).wait()

                @plsc.parallel_loop(0, _HALF, unroll=8)
                def _(r):
                    b1l = lax.shift_right_logical(r, 7)
                    b0 = lax.bitwise_and(r, 127)
                    v0 = rows[r, pl.ds(0, lanes)]
                    v1 = rows[r, pl.ds(lanes, lanes)]
                    i0 = jnp.full((lanes,), b1l, jnp.int32)
                    i2 = clovec + b0
                    plsc.store_scatter(trans, [i0, chivec, i2], v0)
                    plsc.store_scatter(trans, [i0, chivec + 2, i2], v1)

                b1base = b1_hi * 8 + p * 4
                pltpu.async_copy(
                    trans,
                    out_hbm.at[
                        pl.ds(b1base, 4), :, pl.ds(b0_hi * 1024, 1024)
                    ],
                    osem,
                )

                @pl.when(i2 < n_iter - 1)
                def _():
                    gather(h + 2, rows, gsem)

            return 0

        lax.fori_loop(0, n_iter, body, 0)

        for p, (rows, trans, gsem, osem) in enumerate(bufs):
            pltpu.make_async_copy(
                out_hbm.at[pl.ds(0, 4), :, pl.ds(0, 1024)], trans, osem
            ).wait()

    return run(table, ids_native)


def kernel(input_ids, embedding_matrix):
    b, s = input_ids.shape
    ids_native = (
        input_ids.T.astype(jnp.int32)
        .reshape(s // 8, 8, b // 128, 128)
        .transpose(0, 2, 1, 3)
        .reshape(b * s)
    )
    out3 = _sc_embed(ids_native, embedding_matrix)
    return (
        out3.reshape(s, 4, b // 128, 8, 128)
        .transpose(2, 4, 0, 1, 3)
        .reshape(b, s, _DIM)
    )


# speedup vs baseline: 1.9160x; 1.4929x over previous
import functools

import jax
import jax.numpy as jnp
from jax import lax
from jax.experimental import pallas as pl
from jax.experimental.pallas import tpu as pltpu
from jax.experimental.pallas import tpu_sc as plsc

_DIM = 32
_HALF = 512


def _sc_embed(ids_native, table):
    n = ids_native.shape[0]
    info = plsc.get_sparse_core_info()
    nc, ns, lanes = info.num_cores, info.num_subcores, info.num_lanes
    nw = nc * ns
    assert n % nw == 0
    per_w = n // nw
    assert per_w % (2 * _HALF) == 0
    n_half = per_w // _HALF
    n_iter = n_half // 2
    units_per_w = per_w // 1024
    groups = _HALF // lanes

    @functools.partial(
        pl.kernel,
        mesh=plsc.VectorSubcoreMesh(core_axis_name="c", subcore_axis_name="s"),
        out_type=jax.ShapeDtypeStruct(
            (n // 4096, 4, 32, 8, 128), jnp.float32
        ),
        compiler_params=pltpu.CompilerParams(
            use_tc_tiling_on_sc=False, needs_layout_passes=False
        ),
        scratch_types=[
            pltpu.VMEM((per_w,), jnp.int32),
            pltpu.VMEM((_HALF, _DIM), jnp.float32),
            pltpu.VMEM((_HALF, _DIM), jnp.float32),
            pltpu.VMEM((4, 4, 1, 8, 129), jnp.float32),
            pltpu.VMEM((4, 4, 1, 8, 129), jnp.float32),
            pltpu.SemaphoreType.DMA,
            pltpu.SemaphoreType.DMA,
            pltpu.SemaphoreType.DMA,
            pltpu.SemaphoreType.DMA,
        ],
    )
    def run(table_hbm, ids_hbm, out_hbm, idx_all, rows_a, rows_b,
            trans_a, trans_b, ga, gb, oa, ob):
        wid = lax.axis_index("s") * nc + lax.axis_index("c")
        kbase = wid * per_w
        u0 = wid * units_per_w

        pltpu.sync_copy(ids_hbm.at[pl.ds(kbase, per_w)], idx_all)

        iota16 = lax.iota(jnp.int32, lanes)
        chivec = lax.shift_right_logical(iota16, 3)
        clovec = lax.bitwise_and(iota16, 7)
        zerovec = jnp.zeros((lanes,), jnp.int32)
        bufs = ((rows_a, trans_a, ga, oa), (rows_b, trans_b, gb, ob))

        def gather(h, rows, gsem):
            pltpu.async_copy(
                table_hbm.at[idx_all.at[pl.ds(h * _HALF, _HALF)]], rows, gsem
            )

        gather(0, rows_a, ga)
        gather(1, rows_b, gb)

        def body(i2, _):
            for p, (rows, trans, gsem, osem) in enumerate(bufs):
                h = 2 * i2 + p
                u = u0 + i2
                b1_hi = lax.shift_right_logical(u, 5)
                b0_hi = lax.bitwise_and(u, 31)
                hloc = h * _HALF

                pltpu.make_async_copy(
                    table_hbm.at[pl.ds(0, _HALF)], rows, gsem
                ).wait()

                def min_body(g, a):
                    return jnp.minimum(
                        a, idx_all[pl.ds(hloc + g * lanes, lanes)]
                    )

                accv = lax.fori_loop(
                    0, groups, min_body, jnp.full((lanes,), 1, jnp.int32)
                )

                @pl.when(jnp.min(accv) == 0)
                def _():
                    def scan_body(g, _):
                        v = idx_all[pl.ds(hloc + g * lanes, lanes)]
                        m = v == 0
                        cnt = jnp.sum(jnp.where(m, 1, 0))

                        @pl.when(cnt > 0)
                        def _():
                            rvec = g * lanes + iota16
                            zero = jnp.zeros((lanes,), jnp.float32)
                            for j in range(_DIM):
                                cvec = jnp.full((lanes,), j, jnp.int32)
                                plsc.store_scatter(
                                    rows, [rvec, cvec], zero, mask=m
                                )

                        return 0

                    lax.fori_loop(0, groups, scan_body, 0)

                @pl.when(i2 >= 1)
                def _():
                    pltpu.make_async_copy(
                        out_hbm.at[pl.ds(0, 4), :, pl.ds(0, 1), :, :],
                        trans.at[:, :, :, :, pl.ds(0, 128)],
                        osem,
                    ).wait()

                @plsc.parallel_loop(0, _HALF, unroll=8)
                def _(r):
                    b1l = lax.shift_right_logical(r, 7)
                    b0 = lax.bitwise_and(r, 127)
                    v0 = rows[r, pl.ds(0, lanes)]
                    v1 = rows[r, pl.ds(lanes, lanes)]
                    i0 = jnp.full((lanes,), b1l, jnp.int32)
                    i4 = jnp.full((lanes,), b0, jnp.int32)
                    plsc.store_scatter(
                        trans, [i0, chivec, zerovec, clovec, i4], v0
                    )
                    plsc.store_scatter(
                        trans, [i0, chivec + 2, zerovec, clovec, i4], v1
                    )

                b1base = b1_hi * 8 + p * 4
                pltpu.async_copy(
                    trans.at[:, :, :, :, pl.ds(0, 128)],
                    out_hbm.at[
                        pl.ds(b1base, 4), :, pl.ds(b0_hi, 1), :, :
                    ],
                    osem,
                )

                @pl.when(i2 < n_iter - 1)
                def _():
                    gather(h + 2, rows, gsem)

            return 0

        lax.fori_loop(0, n_iter, body, 0)

        for p, (rows, trans, gsem, osem) in enumerate(bufs):
            pltpu.make_async_copy(
                out_hbm.at[pl.ds(0, 4), :, pl.ds(0, 1), :, :],
                trans.at[:, :, :, :, pl.ds(0, 128)],
                osem,
            ).wait()

    return run(table, ids_native)


def kernel(input_ids, embedding_matrix):
    b, s = input_ids.shape
    ids_native = (
        input_ids.T.astype(jnp.int32)
        .reshape(s // 8, 8, b // 128, 128)
        .transpose(0, 2, 1, 3)
        .reshape(b * s)
    )
    out5 = _sc_embed(ids_native, embedding_matrix)
    return out5.transpose(2, 4, 0, 1, 3).reshape(b, s, _DIM)
